# baseline (speedup 1.0000x reference)
import functools

import jax
import jax.numpy as jnp
from jax import lax
from jax.experimental import pallas as pl
from jax.experimental.pallas import tpu as pltpu
from jax.experimental.pallas import tpu_sc as plsc

_V = 1000
_D = 64
_B = 1024
_T = 50
_HT = 25


def _table_body(embed_ref, head_ref, out_ref):
    out_ref[...] = lax.dot_general(
        embed_ref[...], head_ref[...],
        dimension_numbers=(((1,), (1,)), ((), ())),
        preferred_element_type=jnp.float32,
    )


def _make_logits_table(embed_table, head_w):
    return pl.pallas_call(
        _table_body,
        out_shape=jax.ShapeDtypeStruct((_V, _V), jnp.float32),
    )(embed_table, head_w)


def _gather_rows(table, idx):
    info = plsc.get_sparse_core_info()
    nc, ns = info.num_cores, info.num_subcores
    nw = nc * ns
    bpw = _B // nw
    nch = bpw * (_T // _HT)
    rows_per_sub = _V // 8

    mesh = plsc.VectorSubcoreMesh(core_axis_name="c", subcore_axis_name="s")

    @functools.partial(
        pl.kernel,
        out_type=jax.ShapeDtypeStruct((_B, _T, _V), jnp.float32),
        mesh=mesh,
        compiler_params=pltpu.CompilerParams(use_tc_tiling_on_sc=False),
        scratch_types=[
            pltpu.VMEM((2 * bpw, _HT), jnp.int32),
            pltpu.VMEM((2, _HT, _V), jnp.float32),
            pltpu.VMEM_SHARED((_V, _V), jnp.float32),
            pltpu.SemaphoreType.DMA,
            pltpu.SemaphoreType.DMA,
        ],
    )
    def k(table_hbm, idx_hbm, out_hbm, idx_all, rows2, table_sh, gsem, wsem):
        c = lax.axis_index("c")
        s = lax.axis_index("s")
        wid = s * nc + c
        b0 = wid * bpw

        pltpu.sync_copy(idx_hbm.at[pl.ds(2 * b0, 2 * bpw)], idx_all)

        @pl.when(s < 8)
        def _stage():
            r0 = s * rows_per_sub
            pltpu.sync_copy(table_hbm.at[pl.ds(r0, rows_per_sub)],
                            table_sh.at[pl.ds(r0, rows_per_sub)])

        plsc.subcore_barrier()

        def gather(i, buf):
            pltpu.async_copy(table_sh.at[idx_all.at[i]], rows2.at[buf], gsem)

        def wait_gather(buf):
            pltpu.make_async_copy(table_hbm.at[pl.ds(0, _HT)], rows2.at[buf],
                                  gsem).wait()

        def write(i, buf):
            b = lax.div(i, 2)
            t0 = lax.rem(i, 2) * _HT
            pltpu.async_copy(rows2.at[buf],
                             out_hbm.at[b0 + b, pl.ds(t0, _HT)], wsem)

        def wait_write(i, buf):
            b = lax.div(i, 2)
            t0 = lax.rem(i, 2) * _HT
            pltpu.make_async_copy(rows2.at[buf],
                                  out_hbm.at[b0 + b, pl.ds(t0, _HT)],
                                  wsem).wait()

        gather(0, 0)
        gather(1, 1)
        wait_gather(0)
        write(0, 0)

        def body(i, carry):
            p = lax.rem(i, 2)
            q = 1 - p
            wait_write(i - 1, q)
            gather(i + 1, q)
            wait_gather(p)
            write(i, p)
            return carry

        lax.fori_loop(1, nch - 1, body, 0)

        p_last = lax.rem(nch - 1, 2)
        wait_write(nch - 2, 1 - p_last)
        wait_gather(p_last)
        write(nch - 1, p_last)
        wait_write(nch - 1, p_last)

    return k(table, idx)


def kernel(input_ids, embed_table, head_w):
    table = _make_logits_table(embed_table, head_w)
    idx = input_ids.astype(jnp.int32).reshape(_B * _T // _HT, _HT)
    return _gather_rows(table, idx)
